# baseline (speedup 1.0000x reference)
import functools

import jax
import jax.numpy as jnp
from jax import lax
from jax.experimental import pallas as pl
from jax.experimental.pallas import tpu as pltpu
import jax.experimental.pallas.tpu_sc as plsc

N = 10000
E = 320000
D = 128
HC = 64
HL = 64
B = 64
TS = 8
OPC = 16
NCAT = 4
NSZ = 4
NUM_HOPS = 3

NC = 2
NS = 16
NW = NC * NS
EP = E // NW
C = 80
NCH = EP // C
AW = 72
WR = 624
BN = 2048
NB = 5
NP = 10240
BE = 4096
PE = 327680


def _relu(v):
    return jnp.maximum(v, 0.0)


def _mm(a, b):
    return lax.dot_general(a, b, (((1,), (0,)), ((), ())),
                           preferred_element_type=jnp.float32)


def _mm_t(a, b):
    return lax.dot_general(a, b, (((1,), (1,)), ((), ())),
                           preferred_element_type=jnp.float32)


def _proj_body(h_ref, W_ref, A2_ref, xs_ref, as_ref, ad_ref):
    i = pl.program_id(0)
    xs = _mm(h_ref[...], W_ref[...])
    xs_ref[...] = xs
    asad = _mm_t(A2_ref[...], xs)
    as_ref[:, pl.ds(i * BN, BN)] = asad[0:1]
    ad_ref[:, pl.ds(i * BN, BN)] = asad[1:2]


def _run_proj(h, W, A2, din):
    return pl.pallas_call(
        _proj_body,
        grid=(NB,),
        in_specs=[
            pl.BlockSpec((BN, din), lambda i: (i, 0)),
            pl.BlockSpec((din, HC), lambda i: (0, 0)),
            pl.BlockSpec((2, HC), lambda i: (0, 0)),
        ],
        out_specs=[
            pl.BlockSpec((BN, HC), lambda i: (i, 0)),
            pl.BlockSpec((1, NP), lambda i: (0, 0)),
            pl.BlockSpec((1, NP), lambda i: (0, 0)),
        ],
        out_shape=[
            jax.ShapeDtypeStruct((N, HC), jnp.float32),
            jax.ShapeDtypeStruct((1, NP), jnp.float32),
            jax.ShapeDtypeStruct((1, NP), jnp.float32),
        ],
    )(h, W, A2)


def _ae_body(ea_ref, WeS_ref, aeS_ref, o0_ref, o1_ref, o2_ref):
    c3 = jnp.sum(WeS_ref[...] * aeS_ref[...][:, None, :], axis=-1)
    mask = (lax.broadcasted_iota(jnp.int32, (128, 8), 0) // 16
            == lax.broadcasted_iota(jnp.int32, (128, 8), 1))
    ea = ea_ref[...]
    for l, o_ref in ((0, o0_ref), (1, o1_ref), (2, o2_ref)):
        c128 = jnp.concatenate([c3[l]] * 8)[:, None]
        cbig = jnp.where(mask, c128, 0.0)
        o_ref[...] = _mm(ea, cbig)


def _run_ae(ea2, WeS, aeS):
    shp = jax.ShapeDtypeStruct((PE // 8, 8), jnp.float32)
    outs = pl.pallas_call(
        _ae_body,
        grid=(PE // BE,),
        in_specs=[
            pl.BlockSpec((BE // 8, 128), lambda i: (i, 0)),
            pl.BlockSpec((3, 16, HC), lambda i: (0, 0, 0)),
            pl.BlockSpec((3, HC), lambda i: (0, 0)),
        ],
        out_specs=[pl.BlockSpec((BE // 8, 8), lambda i: (i, 0))] * 3,
        out_shape=[shp, shp, shp],
    )(ea2, WeS, aeS)
    return [o.reshape(PE) for o in outs]


def _finproj_body(acc0_ref, acc1_ref, b_ref, W_ref, A2_ref, xs_ref,
                  as_ref, ad_ref):
    i = pl.program_id(0)
    a = acc0_ref[0] + acc1_ref[0]
    h = a[:, :HC] / (a[:, HC:HC + 1] + 1e-16) + b_ref[...]
    h = _relu(h)
    xs = _mm(h, W_ref[...])
    xs_ref[...] = xs
    asad = _mm_t(A2_ref[...], xs)
    as_ref[:, pl.ds(i * BN, BN)] = asad[0:1]
    ad_ref[:, pl.ds(i * BN, BN)] = asad[1:2]


def _run_finproj(acc, b, W, A2):
    return pl.pallas_call(
        _finproj_body,
        grid=(NB,),
        in_specs=[
            pl.BlockSpec((1, BN, AW), lambda i: (0, i, 0)),
            pl.BlockSpec((1, BN, AW), lambda i: (1, i, 0)),
            pl.BlockSpec((1, HC), lambda i: (0, 0)),
            pl.BlockSpec((HC, HC), lambda i: (0, 0)),
            pl.BlockSpec((2, HC), lambda i: (0, 0)),
        ],
        out_specs=[
            pl.BlockSpec((BN, HC), lambda i: (i, 0)),
            pl.BlockSpec((1, NP), lambda i: (0, 0)),
            pl.BlockSpec((1, NP), lambda i: (0, 0)),
        ],
        out_shape=[
            jax.ShapeDtypeStruct((N, HC), jnp.float32),
            jax.ShapeDtypeStruct((1, NP), jnp.float32),
            jax.ShapeDtypeStruct((1, NP), jnp.float32),
        ],
    )(acc, acc, b, W, A2)


def _heads_body(acc0_ref, acc1_ref, gb_ref, x_ref, batch_ref, sW_ref, sb_ref,
                p1W0_ref, p1b0_ref, p1W1_ref, p1b1_ref,
                p2W0_ref, p2b0_ref, p2W1_ref, p2b1_ref,
                p3W0_ref, p3b0_ref, p3W1_ref, p3b1_ref,
                lcW0_ref, lcb0_ref, lcW1_ref, lcb1_ref,
                lsW0_ref, lsb0_ref, lsW1_ref, lsb1_ref,
                lfW0_ref, lfb0_ref, lfW1_ref, lfb1_ref,
                lbW0_ref, lbb0_ref, lbW1_ref, lbb1_ref,
                lzW0_ref, lzb0_ref, lzW1_ref, lzb1_ref,
                p1_ref, p2_ref, p3_ref, lc_ref, ls_ref, lf_ref, lb_ref, lz_ref):
    a = acc0_ref[...] + acc1_ref[...]
    h = a[:, :HC] / (a[:, HC:HC + 1] + 1e-16) + gb_ref[...]

    batch = batch_ref[...]
    bids = lax.broadcasted_iota(jnp.int32, (B, 1), 0)
    idx0 = jnp.sum((batch < bids).astype(jnp.int32), axis=1, keepdims=True)
    niota = lax.broadcasted_iota(jnp.int32, (B, N), 1)
    onehot = (niota == idx0).astype(jnp.float32)
    h0 = _mm(onehot, h)
    dtv = _mm(onehot, x_ref[...])

    sW = sW_ref[...]
    z = _relu(_mm(h0, sW[:HC]) + _mm(dtv, sW[HC:HC + TS]) + sb_ref[...])

    def head(inp, W0_ref, b0_ref, W1_ref, b1_ref):
        hh = _relu(_mm(inp, W0_ref[...]) + b0_ref[...])
        return _mm(hh, W1_ref[...]) + b1_ref[...]

    p1 = head(z, p1W0_ref, p1b0_ref, p1W1_ref, p1b1_ref)
    p2 = head(z, p2W0_ref, p2b0_ref, p2W1_ref, p2b1_ref)
    p3 = head(z, p3W0_ref, p3b0_ref, p3W1_ref, p3b1_ref)
    lc = head(z, lcW0_ref, lcb0_ref, lcW1_ref, lcb1_ref)
    ls = head(z, lsW0_ref, lsb0_ref, lsW1_ref, lsb1_ref)
    lf = head(z, lfW0_ref, lfb0_ref, lfW1_ref, lfb1_ref)

    lbW0 = lbW0_ref[...]
    o = HL
    lb_h = _mm(z, lbW0[:o])
    lb_h += _mm(dtv, lbW0[o:o + TS]); o += TS
    lb_h += _mm(lc, lbW0[o:o + NCAT]); o += NCAT
    lb_h += _mm(ls, lbW0[o:o + 1]); o += 1
    lb_h += _mm(lf, lbW0[o:o + 1])
    lb_h = _relu(lb_h + lbb0_ref[...])
    lb = _mm(lb_h, lbW1_ref[...]) + lbb1_ref[...]

    lzW0 = lzW0_ref[...]
    o = HL
    lz_h = _mm(z, lzW0[:o])
    lz_h += _mm(dtv, lzW0[o:o + TS]); o += TS
    lz_h += _mm(lc, lzW0[o:o + NCAT]); o += NCAT
    lz_h += _mm(ls, lzW0[o:o + 1]); o += 1
    lz_h += _mm(lf, lzW0[o:o + 1]); o += 1
    lz_h += _mm(lb, lzW0[o:o + 1])
    lz_h = _relu(lz_h + lzb0_ref[...])
    lz = _mm(lz_h, lzW1_ref[...]) + lzb1_ref[...]

    p1_ref[...] = p1
    p2_ref[...] = p2
    p3_ref[...] = p3
    lc_ref[...] = lc
    ls_ref[...] = ls
    lf_ref[...] = lf
    lb_ref[...] = lb
    lz_ref[...] = lz


def _run_heads(acc, gb, x_ts, batch, sW, sb, hw):
    return pl.pallas_call(
        _heads_body,
        out_shape=(
            jax.ShapeDtypeStruct((B, 3), jnp.float32),
            jax.ShapeDtypeStruct((B, 3), jnp.float32),
            jax.ShapeDtypeStruct((B, 3), jnp.float32),
            jax.ShapeDtypeStruct((B, NCAT), jnp.float32),
            jax.ShapeDtypeStruct((B, 1), jnp.float32),
            jax.ShapeDtypeStruct((B, 1), jnp.float32),
            jax.ShapeDtypeStruct((B, 1), jnp.float32),
            jax.ShapeDtypeStruct((B, NSZ), jnp.float32),
        ),
    )(acc[0], acc[1], gb, x_ts, batch, sW, sb, *hw)


@functools.partial(
    pl.kernel,
    out_type=jax.ShapeDtypeStruct((2, N, AW), jnp.float32),
    mesh=plsc.VectorSubcoreMesh(core_axis_name="c", subcore_axis_name="s"),
    compiler_params=pltpu.CompilerParams(use_tc_tiling_on_sc=False,
                                         needs_layout_passes=False),
    scratch_types=[
        pltpu.VMEM((1, NP), jnp.float32),
        pltpu.VMEM((1, NP), jnp.float32),
        pltpu.VMEM((C,), jnp.int32),
        pltpu.VMEM((C,), jnp.int32),
        pltpu.VMEM((C,), jnp.int32),
        pltpu.VMEM((C,), jnp.int32),
        pltpu.VMEM((C,), jnp.float32),
        pltpu.VMEM((C,), jnp.float32),
        pltpu.VMEM((C,), jnp.int32),
        pltpu.VMEM((C,), jnp.int32),
        pltpu.VMEM((C,), jnp.float32),
        pltpu.VMEM((C, HC), jnp.float32),
        pltpu.VMEM((C, HC), jnp.float32),
        pltpu.VMEM((C, AW), jnp.float32),
        pltpu.VMEM((C, AW), jnp.float32),
        pltpu.VMEM_SHARED((N, AW), jnp.float32),
        pltpu.SemaphoreType.DMA,
        pltpu.SemaphoreType.DMA,
        pltpu.SemaphoreType.DMA,
        pltpu.SemaphoreType.DMA,
        pltpu.SemaphoreType.DMA,
        pltpu.SemaphoreType.DMA,
    ],
)
def _edge_sc(src_hbm, dst_hbm, ae_hbm, as_hbm, ad_hbm, xs_hbm, out_hbm,
             asb, adb, srcb0, srcb1, dstb0, dstb1, aeb0, aeb1, dsb0, dsb1, wb,
             gb0, gb1, sb0, sb1, acc,
             isem0, isem1, gsem0, gsem1, ssem0, ssem1):
    cid = lax.axis_index("c")
    sid = lax.axis_index("s")
    wid = sid * NC + cid
    ebase = wid * EP

    srcb = [srcb0, srcb1]
    dstb = [dstb0, dstb1]
    aeb = [aeb0, aeb1]
    dsb = [dsb0, dsb1]
    gbuf = [gb0, gb1]
    sbuf = [sb0, sb1]
    isem = [isem0, isem1]
    gsem = [gsem0, gsem1]
    ssem = [ssem0, ssem1]

    lane = lax.iota(jnp.int32, 16)
    zeros16 = jnp.zeros((16,), jnp.float32)
    izeros16 = jnp.zeros((16,), jnp.int32)
    tailmask = lane < (AW - HC)
    NG = C // 16

    def zb(e, _):
        se = jnp.full((16,), e, jnp.int32)
        for b in range(2):
            for k in range(HC // 16):
                plsc.store_scatter(sbuf[b], [se, lane + k * 16], zeros16)
            plsc.store_scatter(sbuf[b], [se, lane + HC], zeros16,
                               mask=tailmask)
        return 0
    lax.fori_loop(0, C, zb, 0)
    for b in range(2):
        for k in range(NG):
            dsb[b][pl.ds(k * 16, 16)] = izeros16

    rbase = sid * WR
    for k in range(7):
        pltpu.sync_copy(sb0, acc.at[pl.ds(rbase + k * C, C)])
    pltpu.sync_copy(sb0.at[pl.ds(0, 64)], acc.at[pl.ds(rbase + 7 * C, 64)])

    @pl.when(sid == NS - 1)
    def _():
        pltpu.sync_copy(sb0.at[pl.ds(0, N - NS * WR)],
                        acc.at[pl.ds(NS * WR, N - NS * WR)])

    pltpu.sync_copy(as_hbm, asb)
    pltpu.sync_copy(ad_hbm, adb)
    plsc.subcore_barrier()

    pltpu.async_copy(sbuf[0], acc.at[dsb[0]], ssem[0], add=True)
    pltpu.async_copy(sbuf[1], acc.at[dsb[1]], ssem[1], add=True)

    def idx_load(gofs, b):
        base = ebase + gofs * C
        pltpu.async_copy(src_hbm.at[pl.ds(base, C)], srcb[b], isem[b])
        pltpu.async_copy(dst_hbm.at[pl.ds(base, C)], dstb[b], isem[b])
        pltpu.async_copy(ae_hbm.at[pl.ds(base, C)], aeb[b], isem[b])

    def idx_wait(b):
        pltpu.make_async_copy(src_hbm.at[pl.ds(0, C)], srcb[b],
                              isem[b]).wait()
        pltpu.make_async_copy(dst_hbm.at[pl.ds(0, C)], dstb[b],
                              isem[b]).wait()
        pltpu.make_async_copy(ae_hbm.at[pl.ds(0, C)], aeb[b], isem[b]).wait()

    izero16 = jnp.zeros((16,), jnp.int32)

    def wpass(b):
        for i in range(NG):
            sl = pl.ds(i * 16, 16)
            av = (plsc.load_gather(asb, [izero16, srcb[b][sl]])
                  + plsc.load_gather(adb, [izero16, dstb[b][sl]])
                  + aeb[b][sl])
            av = jnp.where(av > 0, av, av * 0.2)
            wb[sl] = jnp.exp(av)

    def scale(b):
        for j in range(NG):
            wg = wb[pl.ds(j * 16, 16)]
            for l in range(16):
                e = j * 16 + l
                wsp = jnp.full((16,), wg[l], jnp.float32)
                for k in range(HC // 16):
                    sl = pl.ds(k * 16, 16)
                    sbuf[b][e, sl] = gbuf[b][e, sl] * wsp
                plsc.store_scatter(sbuf[b],
                                   [jnp.full((16,), e, jnp.int32), lane + HC],
                                   jnp.where(lane == 0, wsp, 0.0),
                                   mask=tailmask)

    def steady(g, b):
        wpass(b)
        pltpu.make_async_copy(sbuf[b], acc.at[dsb[b]], ssem[b]).wait()
        pltpu.make_async_copy(xs_hbm.at[srcb[b]], gbuf[b], gsem[b]).wait()
        for k in range(NG):
            sl = pl.ds(k * 16, 16)
            dsb[b][sl] = dstb[b][sl]
        g2 = g + 2
        g2 = jnp.where(g2 >= NCH, g2 - NCH, g2)
        idx_load(g2, b)
        idx_wait(1 - b)
        pltpu.async_copy(xs_hbm.at[srcb[1 - b]], gbuf[1 - b], gsem[1 - b])
        scale(b)
        pltpu.async_copy(sbuf[b], acc.at[dsb[b]], ssem[b], add=True)

    idx_load(0, 0)
    idx_wait(0)
    pltpu.async_copy(xs_hbm.at[srcb[0]], gbuf[0], gsem[0])
    idx_load(1, 1)

    def chunk_loop(g, _):
        @pl.when(g % 2 == 0)
        def _():
            steady(g, 0)

        @pl.when(g % 2 == 1)
        def _():
            steady(g, 1)
        return 0
    lax.fori_loop(0, NCH, chunk_loop, 0)

    pltpu.make_async_copy(sbuf[0], acc.at[dsb[0]], ssem[0]).wait()
    pltpu.make_async_copy(sbuf[1], acc.at[dsb[1]], ssem[1]).wait()
    idx_wait(0)
    pltpu.make_async_copy(xs_hbm.at[srcb[1]], gbuf[1], gsem[1]).wait()

    plsc.subcore_barrier()
    pltpu.sync_copy(acc.at[pl.ds(sid * WR, WR)],
                    out_hbm.at[cid, pl.ds(sid * WR, WR)])

    @pl.when(sid == NS - 1)
    def _():
        pltpu.sync_copy(acc.at[pl.ds(NS * WR, N - NS * WR)],
                        out_hbm.at[cid, pl.ds(NS * WR, N - NS * WR)])


def kernel(x, edge_index, batch, edge_attr,
           gW0, gas0, gad0, gWe0, gae0, gb0,
           gW1, gas1, gad1, gWe1, gae1, gb1,
           gW2, gas2, gad2, gWe2, gae2, gb2,
           sW, sb,
           p1W0, p1b0, p1W1, p1b1,
           p2W0, p2b0, p2W1, p2b1,
           p3W0, p3b0, p3W1, p3b1,
           lcW0, lcb0, lcW1, lcb1,
           lsW0, lsb0, lsW1, lsb1,
           lfW0, lfb0, lfW1, lfb1,
           lbW0, lbb0, lbW1, lbb1,
           lzW0, lzb0, lzW1, lzb1):
    src = edge_index[0]
    dst = edge_index[1]

    A2 = [jnp.concatenate([a_s, a_d], axis=0)
          for a_s, a_d in ((gas0, gad0), (gas1, gad1), (gas2, gad2))]
    WeS = jnp.stack([gWe0, gWe1, gWe2], axis=0)
    aeS = jnp.concatenate([gae0, gae1, gae2], axis=0)

    ea2 = jnp.pad(edge_attr.reshape(E // 8, 128),
                  ((0, PE // 8 - E // 8), (0, 0)))
    ae_all = _run_ae(ea2, WeS, aeS)

    xs, asv, adv = _run_proj(x, gW0, A2[0], D)
    acc = _edge_sc(src, dst, ae_all[0], asv, adv, xs)
    xs, asv, adv = _run_finproj(acc, gb0.reshape(1, HC), gW1, A2[1])
    acc = _edge_sc(src, dst, ae_all[1], asv, adv, xs)
    xs, asv, adv = _run_finproj(acc, gb1.reshape(1, HC), gW2, A2[2])
    acc = _edge_sc(src, dst, ae_all[2], asv, adv, xs)

    hw = (p1W0, p1b0.reshape(1, -1), p1W1, p1b1.reshape(1, -1),
          p2W0, p2b0.reshape(1, -1), p2W1, p2b1.reshape(1, -1),
          p3W0, p3b0.reshape(1, -1), p3W1, p3b1.reshape(1, -1),
          lcW0, lcb0.reshape(1, -1), lcW1, lcb1.reshape(1, -1),
          lsW0, lsb0.reshape(1, -1), lsW1, lsb1.reshape(1, -1),
          lfW0, lfb0.reshape(1, -1), lfW1, lfb1.reshape(1, -1),
          lbW0, lbb0.reshape(1, -1), lbW1, lbb1.reshape(1, -1),
          lzW0, lzb0.reshape(1, -1), lzW1, lzb1.reshape(1, -1))
    x_ts = x[:, OPC:OPC + TS]
    return _run_heads(acc, gb2.reshape(1, HC), x_ts, batch.reshape(1, N),
                      sW, sb.reshape(1, HL), hw)


# speedup vs baseline: 1.0490x; 1.0490x over previous
import functools

import jax
import jax.numpy as jnp
from jax import lax
from jax.experimental import pallas as pl
from jax.experimental.pallas import tpu as pltpu
import jax.experimental.pallas.tpu_sc as plsc

N = 10000
E = 320000
D = 128
HC = 64
HL = 64
B = 64
TS = 8
OPC = 16
NCAT = 4
NSZ = 4
NUM_HOPS = 3

NC = 2
NS = 16
NW = NC * NS
EP = E // NW
C = 80
NCH = EP // C
AW = 72
WR = 624
BN = 2048
NB = 5
NP = 10240
BE = 6400


def _relu(v):
    return jnp.maximum(v, 0.0)


def _mm(a, b):
    return lax.dot_general(a, b, (((1,), (0,)), ((), ())),
                           preferred_element_type=jnp.float32)


def _mm_t(a, b):
    return lax.dot_general(a, b, (((1,), (1,)), ((), ())),
                           preferred_element_type=jnp.float32)


def _proj_body(h_ref, W_ref, A2_ref, xs_ref, as_ref, ad_ref):
    i = pl.program_id(0)
    xs = _mm(h_ref[...], W_ref[...])
    xs_ref[...] = xs
    asad = _mm_t(A2_ref[...], xs)
    as_ref[:, pl.ds(i * BN, BN)] = asad[0:1]
    ad_ref[:, pl.ds(i * BN, BN)] = asad[1:2]


def _run_proj(h, W, A2, din):
    return pl.pallas_call(
        _proj_body,
        grid=(NB,),
        in_specs=[
            pl.BlockSpec((BN, din), lambda i: (i, 0)),
            pl.BlockSpec((din, HC), lambda i: (0, 0)),
            pl.BlockSpec((2, HC), lambda i: (0, 0)),
        ],
        out_specs=[
            pl.BlockSpec((BN, HC), lambda i: (i, 0)),
            pl.BlockSpec((1, NP), lambda i: (0, 0)),
            pl.BlockSpec((1, NP), lambda i: (0, 0)),
        ],
        out_shape=[
            jax.ShapeDtypeStruct((N, HC), jnp.float32),
            jax.ShapeDtypeStruct((1, NP), jnp.float32),
            jax.ShapeDtypeStruct((1, NP), jnp.float32),
        ],
    )(h, W, A2)


def _ae_body(ea_ref, WeS_ref, aeS_ref, out_ref):
    c3 = jnp.sum(WeS_ref[...] * aeS_ref[...][:, None, :], axis=-1)
    c4 = jnp.concatenate([c3, jnp.zeros((1, 16), jnp.float32)], axis=0)
    out_ref[...] = _mm_t(c4, ea_ref[...])


def _run_ae(edge_attr, WeS, aeS):
    return pl.pallas_call(
        _ae_body,
        grid=(E // BE,),
        in_specs=[
            pl.BlockSpec((BE, 16), lambda i: (i, 0)),
            pl.BlockSpec((3, 16, HC), lambda i: (0, 0, 0)),
            pl.BlockSpec((3, HC), lambda i: (0, 0)),
        ],
        out_specs=pl.BlockSpec((4, BE), lambda i: (0, i)),
        out_shape=jax.ShapeDtypeStruct((4, E), jnp.float32),
    )(edge_attr, WeS, aeS)


def _finproj_body(acc0_ref, acc1_ref, b_ref, W_ref, A2_ref, xs_ref,
                  as_ref, ad_ref):
    i = pl.program_id(0)
    a = acc0_ref[0] + acc1_ref[0]
    h = a[:, :HC] / (a[:, HC:HC + 1] + 1e-16) + b_ref[...]
    h = _relu(h)
    xs = _mm(h, W_ref[...])
    xs_ref[...] = xs
    asad = _mm_t(A2_ref[...], xs)
    as_ref[:, pl.ds(i * BN, BN)] = asad[0:1]
    ad_ref[:, pl.ds(i * BN, BN)] = asad[1:2]


def _run_finproj(acc, b, W, A2):
    return pl.pallas_call(
        _finproj_body,
        grid=(NB,),
        in_specs=[
            pl.BlockSpec((1, BN, AW), lambda i: (0, i, 0)),
            pl.BlockSpec((1, BN, AW), lambda i: (1, i, 0)),
            pl.BlockSpec((1, HC), lambda i: (0, 0)),
            pl.BlockSpec((HC, HC), lambda i: (0, 0)),
            pl.BlockSpec((2, HC), lambda i: (0, 0)),
        ],
        out_specs=[
            pl.BlockSpec((BN, HC), lambda i: (i, 0)),
            pl.BlockSpec((1, NP), lambda i: (0, 0)),
            pl.BlockSpec((1, NP), lambda i: (0, 0)),
        ],
        out_shape=[
            jax.ShapeDtypeStruct((N, HC), jnp.float32),
            jax.ShapeDtypeStruct((1, NP), jnp.float32),
            jax.ShapeDtypeStruct((1, NP), jnp.float32),
        ],
    )(acc, acc, b, W, A2)


def _heads_body(acc0_ref, acc1_ref, gb_ref, x_ref, batch_ref, sW_ref, sb_ref,
                p1W0_ref, p1b0_ref, p1W1_ref, p1b1_ref,
                p2W0_ref, p2b0_ref, p2W1_ref, p2b1_ref,
                p3W0_ref, p3b0_ref, p3W1_ref, p3b1_ref,
                lcW0_ref, lcb0_ref, lcW1_ref, lcb1_ref,
                lsW0_ref, lsb0_ref, lsW1_ref, lsb1_ref,
                lfW0_ref, lfb0_ref, lfW1_ref, lfb1_ref,
                lbW0_ref, lbb0_ref, lbW1_ref, lbb1_ref,
                lzW0_ref, lzb0_ref, lzW1_ref, lzb1_ref,
                p1_ref, p2_ref, p3_ref, lc_ref, ls_ref, lf_ref, lb_ref, lz_ref):
    a = acc0_ref[...] + acc1_ref[...]
    h = a[:, :HC] / (a[:, HC:HC + 1] + 1e-16) + gb_ref[...]

    batch = batch_ref[...]
    bids = lax.broadcasted_iota(jnp.int32, (B, 1), 0)
    idx0 = jnp.sum((batch < bids).astype(jnp.int32), axis=1, keepdims=True)
    niota = lax.broadcasted_iota(jnp.int32, (B, N), 1)
    onehot = (niota == idx0).astype(jnp.float32)
    h0 = _mm(onehot, h)
    dtv = _mm(onehot, x_ref[...])

    sW = sW_ref[...]
    z = _relu(_mm(h0, sW[:HC]) + _mm(dtv, sW[HC:HC + TS]) + sb_ref[...])

    def head(inp, W0_ref, b0_ref, W1_ref, b1_ref):
        hh = _relu(_mm(inp, W0_ref[...]) + b0_ref[...])
        return _mm(hh, W1_ref[...]) + b1_ref[...]

    p1 = head(z, p1W0_ref, p1b0_ref, p1W1_ref, p1b1_ref)
    p2 = head(z, p2W0_ref, p2b0_ref, p2W1_ref, p2b1_ref)
    p3 = head(z, p3W0_ref, p3b0_ref, p3W1_ref, p3b1_ref)
    lc = head(z, lcW0_ref, lcb0_ref, lcW1_ref, lcb1_ref)
    ls = head(z, lsW0_ref, lsb0_ref, lsW1_ref, lsb1_ref)
    lf = head(z, lfW0_ref, lfb0_ref, lfW1_ref, lfb1_ref)

    lbW0 = lbW0_ref[...]
    o = HL
    lb_h = _mm(z, lbW0[:o])
    lb_h += _mm(dtv, lbW0[o:o + TS]); o += TS
    lb_h += _mm(lc, lbW0[o:o + NCAT]); o += NCAT
    lb_h += _mm(ls, lbW0[o:o + 1]); o += 1
    lb_h += _mm(lf, lbW0[o:o + 1])
    lb_h = _relu(lb_h + lbb0_ref[...])
    lb = _mm(lb_h, lbW1_ref[...]) + lbb1_ref[...]

    lzW0 = lzW0_ref[...]
    o = HL
    lz_h = _mm(z, lzW0[:o])
    lz_h += _mm(dtv, lzW0[o:o + TS]); o += TS
    lz_h += _mm(lc, lzW0[o:o + NCAT]); o += NCAT
    lz_h += _mm(ls, lzW0[o:o + 1]); o += 1
    lz_h += _mm(lf, lzW0[o:o + 1]); o += 1
    lz_h += _mm(lb, lzW0[o:o + 1])
    lz_h = _relu(lz_h + lzb0_ref[...])
    lz = _mm(lz_h, lzW1_ref[...]) + lzb1_ref[...]

    p1_ref[...] = p1
    p2_ref[...] = p2
    p3_ref[...] = p3
    lc_ref[...] = lc
    ls_ref[...] = ls
    lf_ref[...] = lf
    lb_ref[...] = lb
    lz_ref[...] = lz


def _run_heads(acc, gb, x_ts, batch, sW, sb, hw):
    return pl.pallas_call(
        _heads_body,
        out_shape=(
            jax.ShapeDtypeStruct((B, 3), jnp.float32),
            jax.ShapeDtypeStruct((B, 3), jnp.float32),
            jax.ShapeDtypeStruct((B, 3), jnp.float32),
            jax.ShapeDtypeStruct((B, NCAT), jnp.float32),
            jax.ShapeDtypeStruct((B, 1), jnp.float32),
            jax.ShapeDtypeStruct((B, 1), jnp.float32),
            jax.ShapeDtypeStruct((B, 1), jnp.float32),
            jax.ShapeDtypeStruct((B, NSZ), jnp.float32),
        ),
    )(acc[0], acc[1], gb, x_ts, batch, sW, sb, *hw)


@functools.partial(
    pl.kernel,
    out_type=jax.ShapeDtypeStruct((2, N, AW), jnp.float32),
    mesh=plsc.VectorSubcoreMesh(core_axis_name="c", subcore_axis_name="s"),
    compiler_params=pltpu.CompilerParams(use_tc_tiling_on_sc=False,
                                         needs_layout_passes=False),
    scratch_types=[
        pltpu.VMEM((1, NP), jnp.float32),
        pltpu.VMEM((1, NP), jnp.float32),
        pltpu.VMEM((C,), jnp.int32),
        pltpu.VMEM((C,), jnp.int32),
        pltpu.VMEM((C,), jnp.int32),
        pltpu.VMEM((C,), jnp.int32),
        pltpu.VMEM((C,), jnp.float32),
        pltpu.VMEM((C,), jnp.float32),
        pltpu.VMEM((C,), jnp.int32),
        pltpu.VMEM((C,), jnp.int32),
        pltpu.VMEM((C,), jnp.float32),
        pltpu.VMEM((C, HC), jnp.float32),
        pltpu.VMEM((C, HC), jnp.float32),
        pltpu.VMEM((C, AW), jnp.float32),
        pltpu.VMEM((C, AW), jnp.float32),
        pltpu.VMEM_SHARED((N, AW), jnp.float32),
        pltpu.SemaphoreType.DMA,
        pltpu.SemaphoreType.DMA,
        pltpu.SemaphoreType.DMA,
        pltpu.SemaphoreType.DMA,
        pltpu.SemaphoreType.DMA,
        pltpu.SemaphoreType.DMA,
    ],
)
def _edge_sc(src_hbm, dst_hbm, ae_hbm, as_hbm, ad_hbm, xs_hbm, out_hbm,
             asb, adb, srcb0, srcb1, dstb0, dstb1, aeb0, aeb1, dsb0, dsb1, wb,
             gb0, gb1, sb0, sb1, acc,
             isem0, isem1, gsem0, gsem1, ssem0, ssem1):
    cid = lax.axis_index("c")
    sid = lax.axis_index("s")
    wid = sid * NC + cid
    ebase = wid * EP

    srcb = [srcb0, srcb1]
    dstb = [dstb0, dstb1]
    aeb = [aeb0, aeb1]
    dsb = [dsb0, dsb1]
    gbuf = [gb0, gb1]
    sbuf = [sb0, sb1]
    isem = [isem0, isem1]
    gsem = [gsem0, gsem1]
    ssem = [ssem0, ssem1]

    lane = lax.iota(jnp.int32, 16)
    zeros16 = jnp.zeros((16,), jnp.float32)
    izeros16 = jnp.zeros((16,), jnp.int32)
    tailmask = lane < (AW - HC)
    NG = C // 16

    def zb(e, _):
        se = jnp.full((16,), e, jnp.int32)
        for b in range(2):
            for k in range(HC // 16):
                plsc.store_scatter(sbuf[b], [se, lane + k * 16], zeros16)
            plsc.store_scatter(sbuf[b], [se, lane + HC], zeros16,
                               mask=tailmask)
        return 0
    lax.fori_loop(0, C, zb, 0)
    for b in range(2):
        for k in range(NG):
            dsb[b][pl.ds(k * 16, 16)] = izeros16

    rbase = sid * WR
    for k in range(7):
        pltpu.sync_copy(sb0, acc.at[pl.ds(rbase + k * C, C)])
    pltpu.sync_copy(sb0.at[pl.ds(0, 64)], acc.at[pl.ds(rbase + 7 * C, 64)])

    @pl.when(sid == NS - 1)
    def _():
        pltpu.sync_copy(sb0.at[pl.ds(0, N - NS * WR)],
                        acc.at[pl.ds(NS * WR, N - NS * WR)])

    pltpu.sync_copy(as_hbm, asb)
    pltpu.sync_copy(ad_hbm, adb)
    plsc.subcore_barrier()

    pltpu.async_copy(sbuf[0], acc.at[dsb[0]], ssem[0], add=True)
    pltpu.async_copy(sbuf[1], acc.at[dsb[1]], ssem[1], add=True)

    def idx_load(gofs, b):
        base = ebase + gofs * C
        pltpu.async_copy(src_hbm.at[pl.ds(base, C)], srcb[b], isem[b])
        pltpu.async_copy(dst_hbm.at[pl.ds(base, C)], dstb[b], isem[b])
        pltpu.async_copy(ae_hbm.at[pl.ds(base, C)], aeb[b], isem[b])

    def idx_wait(b):
        pltpu.make_async_copy(src_hbm.at[pl.ds(0, C)], srcb[b],
                              isem[b]).wait()
        pltpu.make_async_copy(dst_hbm.at[pl.ds(0, C)], dstb[b],
                              isem[b]).wait()
        pltpu.make_async_copy(ae_hbm.at[pl.ds(0, C)], aeb[b], isem[b]).wait()

    izero16 = jnp.zeros((16,), jnp.int32)

    def wpass(b):
        for i in range(NG):
            sl = pl.ds(i * 16, 16)
            av = (plsc.load_gather(asb, [izero16, srcb[b][sl]])
                  + plsc.load_gather(adb, [izero16, dstb[b][sl]])
                  + aeb[b][sl])
            av = jnp.where(av > 0, av, av * 0.2)
            wb[sl] = jnp.exp(av)

    def scale(b):
        for j in range(NG):
            wg = wb[pl.ds(j * 16, 16)]
            for l in range(16):
                e = j * 16 + l
                wsp = jnp.full((16,), wg[l], jnp.float32)
                for k in range(HC // 16):
                    sl = pl.ds(k * 16, 16)
                    sbuf[b][e, sl] = gbuf[b][e, sl] * wsp
                plsc.store_scatter(sbuf[b],
                                   [jnp.full((16,), e, jnp.int32), lane + HC],
                                   jnp.where(lane == 0, wsp, 0.0),
                                   mask=tailmask)

    def steady(g, b):
        wpass(b)
        pltpu.make_async_copy(sbuf[b], acc.at[dsb[b]], ssem[b]).wait()
        pltpu.make_async_copy(xs_hbm.at[srcb[b]], gbuf[b], gsem[b]).wait()
        for k in range(NG):
            sl = pl.ds(k * 16, 16)
            dsb[b][sl] = dstb[b][sl]
        g2 = g + 2
        g2 = jnp.where(g2 >= NCH, g2 - NCH, g2)
        idx_load(g2, b)
        idx_wait(1 - b)
        pltpu.async_copy(xs_hbm.at[srcb[1 - b]], gbuf[1 - b], gsem[1 - b])
        scale(b)
        pltpu.async_copy(sbuf[b], acc.at[dsb[b]], ssem[b], add=True)

    idx_load(0, 0)
    idx_wait(0)
    pltpu.async_copy(xs_hbm.at[srcb[0]], gbuf[0], gsem[0])
    idx_load(1, 1)

    def chunk_loop(g, _):
        @pl.when(g % 2 == 0)
        def _():
            steady(g, 0)

        @pl.when(g % 2 == 1)
        def _():
            steady(g, 1)
        return 0
    lax.fori_loop(0, NCH, chunk_loop, 0)

    pltpu.make_async_copy(sbuf[0], acc.at[dsb[0]], ssem[0]).wait()
    pltpu.make_async_copy(sbuf[1], acc.at[dsb[1]], ssem[1]).wait()
    idx_wait(0)
    pltpu.make_async_copy(xs_hbm.at[srcb[1]], gbuf[1], gsem[1]).wait()

    plsc.subcore_barrier()
    pltpu.sync_copy(acc.at[pl.ds(sid * WR, WR)],
                    out_hbm.at[cid, pl.ds(sid * WR, WR)])

    @pl.when(sid == NS - 1)
    def _():
        pltpu.sync_copy(acc.at[pl.ds(NS * WR, N - NS * WR)],
                        out_hbm.at[cid, pl.ds(NS * WR, N - NS * WR)])


def kernel(x, edge_index, batch, edge_attr,
           gW0, gas0, gad0, gWe0, gae0, gb0,
           gW1, gas1, gad1, gWe1, gae1, gb1,
           gW2, gas2, gad2, gWe2, gae2, gb2,
           sW, sb,
           p1W0, p1b0, p1W1, p1b1,
           p2W0, p2b0, p2W1, p2b1,
           p3W0, p3b0, p3W1, p3b1,
           lcW0, lcb0, lcW1, lcb1,
           lsW0, lsb0, lsW1, lsb1,
           lfW0, lfb0, lfW1, lfb1,
           lbW0, lbb0, lbW1, lbb1,
           lzW0, lzb0, lzW1, lzb1):
    src = edge_index[0]
    dst = edge_index[1]

    A2 = [jnp.concatenate([a_s, a_d], axis=0)
          for a_s, a_d in ((gas0, gad0), (gas1, gad1), (gas2, gad2))]
    WeS = jnp.stack([gWe0, gWe1, gWe2], axis=0)
    aeS = jnp.concatenate([gae0, gae1, gae2], axis=0)

    ae_all = _run_ae(edge_attr, WeS, aeS)

    xs, asv, adv = _run_proj(x, gW0, A2[0], D)
    acc = _edge_sc(src, dst, ae_all[0], asv, adv, xs)
    xs, asv, adv = _run_finproj(acc, gb0.reshape(1, HC), gW1, A2[1])
    acc = _edge_sc(src, dst, ae_all[1], asv, adv, xs)
    xs, asv, adv = _run_finproj(acc, gb1.reshape(1, HC), gW2, A2[2])
    acc = _edge_sc(src, dst, ae_all[2], asv, adv, xs)

    hw = (p1W0, p1b0.reshape(1, -1), p1W1, p1b1.reshape(1, -1),
          p2W0, p2b0.reshape(1, -1), p2W1, p2b1.reshape(1, -1),
          p3W0, p3b0.reshape(1, -1), p3W1, p3b1.reshape(1, -1),
          lcW0, lcb0.reshape(1, -1), lcW1, lcb1.reshape(1, -1),
          lsW0, lsb0.reshape(1, -1), lsW1, lsb1.reshape(1, -1),
          lfW0, lfb0.reshape(1, -1), lfW1, lfb1.reshape(1, -1),
          lbW0, lbb0.reshape(1, -1), lbW1, lbb1.reshape(1, -1),
          lzW0, lzb0.reshape(1, -1), lzW1, lzb1.reshape(1, -1))
    x_ts = x[:, OPC:OPC + TS]
    return _run_heads(acc, gb2.reshape(1, HC), x_ts, batch.reshape(1, N),
                      sW, sb.reshape(1, HL), hw)
